# baseline (speedup 1.0000x reference)
import jax
import jax.numpy as jnp
from jax import lax
from jax.experimental import pallas as pl
from jax.experimental.pallas import tpu as pltpu
from jax.experimental.pallas import tpu_sc as plsc

VOCAB = 100000
D = 1024
MAX_LEN = 2048
B = 4
ROWS = B * MAX_LEN
NC, NS, L = 2, 16, 16
NW = NC * NS
TPW = MAX_LEN // NW
SCT = 8
NSC = TPW // SCT
CR = B * SCT
NSLOT = 3
GROUPS = D // L
BLK = 4


def _body(tok_hbm, seg_hbm, table_hbm, segtab_hbm, pos_hbm, out_hbm,
          tokv, segv, rows, posc, segtab, isem, gsem, ssem, psem):
    cid = lax.axis_index("c")
    sid = lax.axis_index("s")
    wid = sid * NC + cid
    t0 = wid * TPW
    lane = lax.iota(jnp.int32, L)

    idx_cps = []
    for b in range(B):
        idx_cps.append(pltpu.async_copy(
            tok_hbm.at[pl.ds(b * MAX_LEN + t0, TPW)], tokv.at[b], isem))
        idx_cps.append(pltpu.async_copy(
            seg_hbm.at[pl.ds(b * MAX_LEN + t0, TPW)], segv.at[b], isem))
    idx_cps.append(pltpu.async_copy(segtab_hbm, segtab, isem))
    for cp in idx_cps:
        cp.wait()

    def gathers(i, slot):
        for b in range(B):
            pltpu.async_copy(
                table_hbm.at[tokv.at[b, pl.ds(i * SCT, SCT)]],
                rows.at[slot, pl.ds(b * SCT, SCT)], gsem.at[slot])

    def wait_gathers(i, slot):
        for b in range(B):
            pltpu.make_async_copy(
                table_hbm.at[tokv.at[b, pl.ds(i * SCT, SCT)]],
                rows.at[slot, pl.ds(b * SCT, SCT)], gsem.at[slot]).wait()

    def pos_cp(i, pslot):
        return pltpu.make_async_copy(
            pos_hbm.at[pl.ds(t0 + i * SCT, SCT)], posc.at[pslot], psem.at[pslot])

    def store(i, slot, b):
        return pltpu.make_async_copy(
            rows.at[slot, pl.ds(b * SCT, SCT)],
            out_hbm.at[pl.ds(b * MAX_LEN + t0 + i * SCT, SCT)],
            ssem.at[slot])

    def wait_stores(i, slot):
        for b in range(B):
            store(i, slot, b).wait()

    gathers(0, 0)
    gathers(1, 1)
    pos_cp(0, 0).start()
    pos_cp(1, 1).start()

    def chunk_body(i, _):
        slot = i % NSLOT
        pslot = i % 2

        @pl.when(jnp.logical_and(i >= 1, i + 1 < NSC))
        def _prefetch():
            nslot = (i + 1) % NSLOT

            @pl.when(i >= 2)
            def _drain():
                wait_stores(i - 2, nslot)
            gathers(i + 1, nslot)

            @pl.when(i >= 1)
            def _pos_next():
                pos_cp(i + 1, (i + 1) % 2).start()

        wait_gathers(i, slot)
        pos_cp(i, pslot).wait()

        half = i // 2
        lpos0 = (i % 2) * SCT

        def pos_body(k, _):
            tl16 = jnp.zeros((L,), jnp.int32) + (i * SCT + k)
            masks = []
            for b in range(B):
                splat = plsc.load_gather(
                    segv, [jnp.full((L,), b, jnp.int32), tl16])
                masks.append(splat == 1)

            for j0 in range(0, GROUPS, BLK):
                sls = [pl.ds((j0 + j) * L, L) for j in range(BLK)]
                pv = [posc[pslot, k, sl] for sl in sls]
                s0 = [segtab[0, sl] for sl in sls]
                s1 = [segtab[1, sl] for sl in sls]
                rv = [[rows[slot, b * SCT + k, sl] for sl in sls]
                      for b in range(B)]
                a0 = [pv[j] + s0[j] for j in range(BLK)]
                a1 = [pv[j] + s1[j] for j in range(BLK)]
                for b in range(B):
                    for j in range(BLK):
                        rows[slot, b * SCT + k, sls[j]] = (
                            rv[b][j] + jnp.where(masks[b], a1[j], a0[j]))
            return 0

        lax.fori_loop(0, SCT, pos_body, 0)

        for b in range(B):
            store(i, slot, b).start()
        return 0

    lax.fori_loop(0, NSC, chunk_body, 0)
    wait_stores(NSC - 1, (NSC - 1) % NSLOT)
    wait_stores(NSC - 2, (NSC - 2) % NSLOT)


@jax.jit
def _run(tokens_flat, segments_flat, token_table, segment_table, pos_w):
    mesh = plsc.VectorSubcoreMesh(core_axis_name="c", subcore_axis_name="s",
                                  num_cores=NC, num_subcores=NS)
    return pl.kernel(
        _body,
        out_type=jax.ShapeDtypeStruct((ROWS, D), jnp.float32),
        mesh=mesh,
        scratch_types=[
            pltpu.VMEM((B, TPW), jnp.int32),
            pltpu.VMEM((B, TPW), jnp.int32),
            pltpu.VMEM((NSLOT, CR, D), jnp.float32),
            pltpu.VMEM((2, SCT, D), jnp.float32),
            pltpu.VMEM((2, D), jnp.float32),
            pltpu.SemaphoreType.DMA,
            pltpu.SemaphoreType.DMA((NSLOT,)),
            pltpu.SemaphoreType.DMA((NSLOT,)),
            pltpu.SemaphoreType.DMA((2,)),
        ],
        compiler_params=pltpu.CompilerParams(
            needs_layout_passes=False,
            skip_device_barrier=True,
            disable_bounds_checks=True,
            disable_semaphore_checks=True,
        ),
    )(tokens_flat, segments_flat, token_table, segment_table, pos_w)


def kernel(tokens, segments, token_table, segment_table, pos_w):
    tokens_flat = tokens.reshape(ROWS).astype(jnp.int32)
    segments_flat = segments.reshape(ROWS).astype(jnp.int32)
    out = _run(tokens_flat, segments_flat, token_table, segment_table, pos_w)
    return out.reshape(B, MAX_LEN, D)


# speedup vs baseline: 1.0066x; 1.0066x over previous
import jax
import jax.numpy as jnp
from jax import lax
from jax.experimental import pallas as pl
from jax.experimental.pallas import tpu as pltpu
from jax.experimental.pallas import tpu_sc as plsc

VOCAB = 100000
D = 1024
MAX_LEN = 2048
B = 4
ROWS = B * MAX_LEN
NC, NS, L = 2, 16, 16
NW = NC * NS
TPW = MAX_LEN // NW
SCT = 8
NSC = TPW // SCT
CR = B * SCT
NSLOT = 3
GROUPS = D // L
BLK = 4


def _body(tok_hbm, seg_hbm, table_hbm, segtab_hbm, pos_hbm, out_hbm,
          tokv, segv, rows, posc, segtab, isem, gsem, ssem, psem):
    cid = lax.axis_index("c")
    sid = lax.axis_index("s")
    wid = sid * NC + cid
    t0 = wid * TPW
    lane = lax.iota(jnp.int32, L)

    idx_cps = []
    for b in range(B):
        idx_cps.append(pltpu.async_copy(
            tok_hbm.at[pl.ds(b * MAX_LEN + t0, TPW)], tokv.at[b], isem))
        idx_cps.append(pltpu.async_copy(
            seg_hbm.at[pl.ds(b * MAX_LEN + t0, TPW)], segv.at[b], isem))
    idx_cps.append(pltpu.async_copy(segtab_hbm, segtab, isem))
    for cp in idx_cps:
        cp.wait()

    def gathers(i, slot):
        for b in range(B):
            pltpu.async_copy(
                table_hbm.at[tokv.at[b, pl.ds(i * SCT, SCT)]],
                rows.at[slot, pl.ds(b * SCT, SCT)], gsem.at[slot])

    def wait_gathers(i, slot):
        for b in range(B):
            pltpu.make_async_copy(
                table_hbm.at[tokv.at[b, pl.ds(i * SCT, SCT)]],
                rows.at[slot, pl.ds(b * SCT, SCT)], gsem.at[slot]).wait()

    def pos_cp(i, pslot):
        return pltpu.make_async_copy(
            pos_hbm.at[pl.ds(t0 + i * SCT, SCT)], posc.at[pslot], psem.at[pslot])

    def store(i, slot, b):
        return pltpu.make_async_copy(
            rows.at[slot, pl.ds(b * SCT, SCT)],
            out_hbm.at[pl.ds(b * MAX_LEN + t0 + i * SCT, SCT)],
            ssem.at[slot])

    def wait_stores(i, slot):
        for b in range(B):
            store(i, slot, b).wait()

    gathers(0, 0)
    gathers(1, 1)
    pos_cp(0, 0).start()
    pos_cp(1, 1).start()

    def chunk_body(i, _):
        slot = i % NSLOT
        pslot = i % 2

        @pl.when(jnp.logical_and(i >= 1, i + 1 < NSC))
        def _prefetch():
            nslot = (i + 1) % NSLOT

            @pl.when(i >= 2)
            def _drain():
                wait_stores(i - 2, nslot)
            gathers(i + 1, nslot)

            @pl.when(i >= 1)
            def _pos_next():
                pos_cp(i + 1, (i + 1) % 2).start()

        wait_gathers(i, slot)
        pos_cp(i, pslot).wait()

        half = i // 2
        lpos0 = (i % 2) * SCT

        def pos_body(k, _):
            tl16 = jnp.zeros((L,), jnp.int32) + (i * SCT + k)
            masks = []
            for b in range(B):
                splat = plsc.load_gather(
                    segv, [jnp.full((L,), b, jnp.int32), tl16])
                masks.append(splat == 1)

            for j0 in range(0, GROUPS, BLK):
                sls = [pl.ds((j0 + j) * L, L) for j in range(BLK)]
                pv = [posc[pslot, k, sl] for sl in sls]
                s0 = [segtab[0, sl] for sl in sls]
                s1 = [segtab[1, sl] for sl in sls]
                a0 = [pv[j] + s0[j] for j in range(BLK)]
                a1 = [pv[j] + s1[j] for j in range(BLK)]
                for b in range(B):
                    for j in range(BLK):
                        plsc.addupdate(
                            rows.at[slot, b * SCT + k, sls[j]],
                            jnp.where(masks[b], a1[j], a0[j]))
            return 0

        lax.fori_loop(0, SCT, pos_body, 0)

        for b in range(B):
            store(i, slot, b).start()
        return 0

    lax.fori_loop(0, NSC, chunk_body, 0)
    wait_stores(NSC - 1, (NSC - 1) % NSLOT)
    wait_stores(NSC - 2, (NSC - 2) % NSLOT)


@jax.jit
def _run(tokens_flat, segments_flat, token_table, segment_table, pos_w):
    mesh = plsc.VectorSubcoreMesh(core_axis_name="c", subcore_axis_name="s",
                                  num_cores=NC, num_subcores=NS)
    return pl.kernel(
        _body,
        out_type=jax.ShapeDtypeStruct((ROWS, D), jnp.float32),
        mesh=mesh,
        scratch_types=[
            pltpu.VMEM((B, TPW), jnp.int32),
            pltpu.VMEM((B, TPW), jnp.int32),
            pltpu.VMEM((NSLOT, CR, D), jnp.float32),
            pltpu.VMEM((2, SCT, D), jnp.float32),
            pltpu.VMEM((2, D), jnp.float32),
            pltpu.SemaphoreType.DMA,
            pltpu.SemaphoreType.DMA((NSLOT,)),
            pltpu.SemaphoreType.DMA((NSLOT,)),
            pltpu.SemaphoreType.DMA((2,)),
        ],
        compiler_params=pltpu.CompilerParams(
            needs_layout_passes=False,
            skip_device_barrier=True,
            disable_bounds_checks=True,
            disable_semaphore_checks=True,
        ),
    )(tokens_flat, segments_flat, token_table, segment_table, pos_w)


def kernel(tokens, segments, token_table, segment_table, pos_w):
    tokens_flat = tokens.reshape(ROWS).astype(jnp.int32)
    segments_flat = segments.reshape(ROWS).astype(jnp.int32)
    out = _run(tokens_flat, segments_flat, token_table, segment_table, pos_w)
    return out.reshape(B, MAX_LEN, D)
